# baseline (speedup 1.0000x reference)
import jax
import jax.numpy as jnp
from jax.experimental import pallas as pl
from jax.experimental.pallas import tpu as pltpu

_EPS = 1e-4
_BM = 256


def _ff_body(x_ref, b_ref, w_hbm, o_ref, w_vmem, r_buf, sem):
    m = pl.program_id(0)

    @pl.when(m == 0)
    def _load_w():
        cp = pltpu.make_async_copy(w_hbm, w_vmem, sem)
        cp.start()
        cp.wait()

    xb = x_ref[...].astype(jnp.bfloat16)
    acc = jax.lax.dot_general(
        xb, w_vmem[...], (((1,), (1,)), ((), ())),
        preferred_element_type=jnp.float32,
    )
    r_buf[jax.lax.rem(m, 2)] = jnp.maximum(acc + b_ref[...], 0.0) + x_ref[...]

    r = r_buf[jax.lax.rem(m + 1, 2)]
    mu = jnp.mean(r, axis=-1, keepdims=True)
    d = r - mu
    v = jnp.mean(d * d, axis=-1, keepdims=True)
    o_ref[...] = d / jnp.sqrt(v + _EPS)


@jax.jit
def kernel(x, w1, b1):
    n, f = x.shape
    w_bf = w1.astype(jnp.bfloat16)
    b2d = b1.reshape(1, f)

    nm = n // _BM
    grid = (nm + 1,)
    return pl.pallas_call(
        _ff_body,
        grid=grid,
        in_specs=[
            pl.BlockSpec((_BM, f), lambda m: (jnp.minimum(m, nm - 1), 0)),
            pl.BlockSpec((1, f), lambda m: (0, 0)),
            pl.BlockSpec(memory_space=pl.ANY),
        ],
        out_specs=pl.BlockSpec((_BM, f), lambda m: (jnp.maximum(m - 1, 0), 0)),
        out_shape=jax.ShapeDtypeStruct((n, f), jnp.float32),
        scratch_shapes=[
            pltpu.VMEM((f, f), jnp.bfloat16),
            pltpu.VMEM((2, _BM, f), jnp.float32),
            pltpu.SemaphoreType.DMA,
        ],
        compiler_params=pltpu.CompilerParams(
            dimension_semantics=("arbitrary",),
            vmem_limit_bytes=63 * 1024 * 1024,
        ),
    )(x, b2d, w_bf)


# speedup vs baseline: 1.0154x; 1.0154x over previous
import jax
import jax.numpy as jnp
from jax.experimental import pallas as pl
from jax.experimental.pallas import tpu as pltpu

_EPS = 1e-4
_BM = 256


def _ff_body(x_ref, b_ref, w_hbm, o_ref, w_vmem, sem):
    @pl.when(pl.program_id(0) == 0)
    def _load_w():
        cp = pltpu.make_async_copy(w_hbm, w_vmem, sem)
        cp.start()
        cp.wait()

    xb = x_ref[...].astype(jnp.bfloat16)
    acc = jax.lax.dot_general(
        xb, w_vmem[...], (((1,), (1,)), ((), ())),
        preferred_element_type=jnp.float32,
    )
    r = jnp.maximum(acc + b_ref[...], 0.0) + x_ref[...]
    mu = jnp.mean(r, axis=-1, keepdims=True)
    d = r - mu
    v = jnp.mean(d * d, axis=-1, keepdims=True)
    o_ref[...] = d / jnp.sqrt(v + _EPS)


@jax.jit
def kernel(x, w1, b1):
    n, f = x.shape
    w_bf = w1.astype(jnp.bfloat16)
    b2d = b1.reshape(1, f)

    grid = (n // _BM,)
    return pl.pallas_call(
        _ff_body,
        grid=grid,
        in_specs=[
            pl.BlockSpec((_BM, f), lambda m: (m, 0)),
            pl.BlockSpec((1, f), lambda m: (0, 0)),
            pl.BlockSpec(memory_space=pl.ANY),
        ],
        out_specs=pl.BlockSpec((_BM, f), lambda m: (m, 0)),
        out_shape=jax.ShapeDtypeStruct((n, f), jnp.float32),
        scratch_shapes=[
            pltpu.VMEM((f, f), jnp.bfloat16),
            pltpu.SemaphoreType.DMA,
        ],
        compiler_params=pltpu.CompilerParams(
            dimension_semantics=("arbitrary",),
            vmem_limit_bytes=63 * 1024 * 1024,
        ),
    )(x, b2d, w_bf)


# speedup vs baseline: 1.0424x; 1.0266x over previous
import jax
import jax.numpy as jnp
from jax.experimental import pallas as pl
from jax.experimental.pallas import tpu as pltpu

_EPS = 1e-4
_BM = 256


def _ff_body(x_ref, b_ref, w_hbm, o_ref, w_vmem, sem):
    @pl.when(pl.program_id(0) == 0)
    def _load_w():
        cp = pltpu.make_async_copy(w_hbm, w_vmem, sem)
        cp.start()
        cp.wait()

    xb = x_ref[...].astype(jnp.bfloat16)
    acc = jax.lax.dot_general(
        xb, w_vmem[...], (((1,), (1,)), ((), ())),
        preferred_element_type=jnp.float32,
    )
    r = jnp.maximum(acc + b_ref[...], 0.0) + x_ref[...]
    inv_f = 1.0 / r.shape[-1]
    mu = jnp.sum(r, axis=-1, keepdims=True) * inv_f
    ms = jnp.sum(r * r, axis=-1, keepdims=True) * inv_f
    v = ms - mu * mu
    o_ref[...] = (r - mu) / jnp.sqrt(v + _EPS)


@jax.jit
def kernel(x, w1, b1):
    n, f = x.shape
    w_bf = w1.astype(jnp.bfloat16)
    b2d = b1.reshape(1, f)

    grid = (n // _BM,)
    return pl.pallas_call(
        _ff_body,
        grid=grid,
        in_specs=[
            pl.BlockSpec((_BM, f), lambda m: (m, 0)),
            pl.BlockSpec((1, f), lambda m: (0, 0)),
            pl.BlockSpec(memory_space=pl.ANY),
        ],
        out_specs=pl.BlockSpec((_BM, f), lambda m: (m, 0)),
        out_shape=jax.ShapeDtypeStruct((n, f), jnp.float32),
        scratch_shapes=[
            pltpu.VMEM((f, f), jnp.bfloat16),
            pltpu.SemaphoreType.DMA,
        ],
        compiler_params=pltpu.CompilerParams(
            dimension_semantics=("arbitrary",),
            vmem_limit_bytes=63 * 1024 * 1024,
        ),
    )(x, b2d, w_bf)


# speedup vs baseline: 1.0754x; 1.0316x over previous
import jax
import jax.numpy as jnp
from jax.experimental import pallas as pl
from jax.experimental.pallas import tpu as pltpu

_EPS = 1e-4
_BM = 256
_WCHUNK = 256


def _ff_body(x_ref, b_ref, w_hbm, o_ref, w_vmem, stage, sems):
    @pl.when(pl.program_id(0) == 0)
    def _load_w():
        nch = w_vmem.shape[0] // _WCHUNK

        def cp(i):
            return pltpu.make_async_copy(
                w_hbm.at[pl.ds(i * _WCHUNK, _WCHUNK), :],
                stage.at[i % 2],
                sems.at[i % 2],
            )

        cp(0).start()
        for i in range(nch):
            if i + 1 < nch:
                cp(i + 1).start()
            cp(i).wait()
            w_vmem[pl.ds(i * _WCHUNK, _WCHUNK), :] = (
                stage[i % 2].astype(jnp.bfloat16))

    xb = x_ref[...].astype(jnp.bfloat16)
    acc = jax.lax.dot_general(
        xb, w_vmem[...], (((1,), (1,)), ((), ())),
        preferred_element_type=jnp.float32,
    )
    r = jnp.maximum(acc + b_ref[...], 0.0) + x_ref[...]
    inv_f = 1.0 / r.shape[-1]
    mu = jnp.sum(r, axis=-1, keepdims=True) * inv_f
    ms = jnp.sum(r * r, axis=-1, keepdims=True) * inv_f
    v = ms - mu * mu
    o_ref[...] = (r - mu) / jnp.sqrt(v + _EPS)


@jax.jit
def kernel(x, w1, b1):
    n, f = x.shape
    b2d = b1.reshape(1, f)

    grid = (n // _BM,)
    return pl.pallas_call(
        _ff_body,
        grid=grid,
        in_specs=[
            pl.BlockSpec((_BM, f), lambda m: (m, 0)),
            pl.BlockSpec((1, f), lambda m: (0, 0)),
            pl.BlockSpec(memory_space=pl.ANY),
        ],
        out_specs=pl.BlockSpec((_BM, f), lambda m: (m, 0)),
        out_shape=jax.ShapeDtypeStruct((n, f), jnp.float32),
        scratch_shapes=[
            pltpu.VMEM((f, f), jnp.bfloat16),
            pltpu.VMEM((2, _WCHUNK, f), jnp.float32),
            pltpu.SemaphoreType.DMA((2,)),
        ],
        compiler_params=pltpu.CompilerParams(
            dimension_semantics=("arbitrary",),
            vmem_limit_bytes=63 * 1024 * 1024,
        ),
    )(x, b2d, w1)
